# baseline (speedup 1.0000x reference)
import functools

import jax
import jax.numpy as jnp
from jax import lax
from jax.experimental import pallas as pl
from jax.experimental.pallas import tpu as pltpu
from jax.experimental.pallas import tpu_sc as plsc

D_MODEL = 128
NUM_CORES = 2
NUM_SUBCORES = 16
NUM_WORKERS = NUM_CORES * NUM_SUBCORES
CHUNK = 128


def _project_table_body(table_ref, w_ref, b_ref, out_ref):
    out_ref[...] = (
        jnp.dot(table_ref[...], w_ref[...], preferred_element_type=jnp.float32)
        + b_ref[...]
    )


def _make_gather(t_total: int):
    assert t_total % (NUM_WORKERS * CHUNK) == 0
    chunks_per_worker = t_total // (NUM_WORKERS * CHUNK)
    mesh = plsc.VectorSubcoreMesh(
        core_axis_name="c",
        subcore_axis_name="s",
        num_cores=NUM_CORES,
        num_subcores=NUM_SUBCORES,
    )

    assert chunks_per_worker % 4 == 0

    @functools.partial(
        pl.kernel,
        out_type=jax.ShapeDtypeStruct((t_total, D_MODEL), jnp.float32),
        mesh=mesh,
        scratch_types=[
            pltpu.VMEM_SHARED((64, D_MODEL), jnp.float32),
            pltpu.VMEM((chunks_per_worker, CHUNK), jnp.int32),
            [pltpu.VMEM((CHUNK, D_MODEL), jnp.float32) for _ in range(4)],
            [pltpu.SemaphoreType.DMA for _ in range(2)],
            [pltpu.SemaphoreType.DMA for _ in range(2)],
        ],
    )
    def gather_kernel(p_hbm, tok_hbm, out_hbm, p_sh, idx_v, rows, gsems, wsems):
        wid = lax.axis_index("s") * NUM_CORES + lax.axis_index("c")
        chunk0 = wid * chunks_per_worker
        @pl.when(lax.axis_index("s") == 0)
        def _():
            pltpu.sync_copy(p_hbm, p_sh)

        pltpu.sync_copy(tok_hbm.at[pl.ds(chunk0, chunks_per_worker)], idx_v)
        plsc.subcore_barrier()

        def out_at(j):
            return out_hbm.at[pl.ds((chunk0 + j) * CHUNK, CHUNK)]

        @pl.loop(0, chunks_per_worker, step=4)
        def _(j):
            descs = []
            for pair in range(2):
                @pl.when(j >= 4)
                def _():
                    for b in range(2):
                        pltpu.make_async_copy(
                            rows[2 * pair + b], out_at(0), wsems[pair]
                        ).wait()
                descs.append([
                    pltpu.async_copy(
                        p_sh.at[idx_v.at[j + 2 * pair + b]],
                        rows[2 * pair + b],
                        gsems[pair],
                    )
                    for b in range(2)
                ])
            for pair in range(2):
                for b in range(2):
                    descs[pair][b].wait()
                    pltpu.async_copy(
                        rows[2 * pair + b], out_at(j + 2 * pair + b), wsems[pair]
                    )

        for pair in range(2):
            for b in range(2):
                pltpu.make_async_copy(rows[2 * pair + b], out_at(0), wsems[pair]).wait()

    return gather_kernel


def kernel(tokens, embedding_table, W, b):
    k, fan_in = embedding_table.shape
    t_total = tokens.shape[0]
    p = embedding_table @ W + b[None, :]
    tok2d = tokens.astype(jnp.int32).reshape(t_total // CHUNK, CHUNK)
    return _make_gather(t_total)(p, tok2d)


# speedup vs baseline: 1.0014x; 1.0014x over previous
import functools

import jax
import jax.numpy as jnp
from jax import lax
from jax.experimental import pallas as pl
from jax.experimental.pallas import tpu as pltpu
from jax.experimental.pallas import tpu_sc as plsc

D_MODEL = 128
NUM_CORES = 2
NUM_SUBCORES = 16
NUM_WORKERS = NUM_CORES * NUM_SUBCORES
CHUNK = 128


def _project_table_body(table_ref, w_ref, b_ref, out_ref):
    out_ref[...] = (
        jnp.dot(table_ref[...], w_ref[...], preferred_element_type=jnp.float32)
        + b_ref[...]
    )


def _make_gather(t_total: int):
    assert t_total % (NUM_WORKERS * CHUNK) == 0
    chunks_per_worker = t_total // (NUM_WORKERS * CHUNK)
    mesh = plsc.VectorSubcoreMesh(
        core_axis_name="c",
        subcore_axis_name="s",
        num_cores=NUM_CORES,
        num_subcores=NUM_SUBCORES,
    )

    assert chunks_per_worker % 4 == 0

    @functools.partial(
        pl.kernel,
        out_type=jax.ShapeDtypeStruct((t_total, D_MODEL), jnp.float32),
        mesh=mesh,
        scratch_types=[
            pltpu.VMEM_SHARED((64, D_MODEL), jnp.float32),
            pltpu.VMEM((chunks_per_worker, CHUNK), jnp.int32),
            [pltpu.VMEM((CHUNK, D_MODEL), jnp.float32) for _ in range(4)],
            [pltpu.SemaphoreType.DMA for _ in range(2)],
            [pltpu.SemaphoreType.DMA for _ in range(2)],
        ],
    )
    def gather_kernel(p_hbm, tok_hbm, out_hbm, p_sh, idx_v, rows, gsems, wsems):
        wid = lax.axis_index("s") * NUM_CORES + lax.axis_index("c")
        chunk0 = wid * chunks_per_worker
        @pl.when(lax.axis_index("s") == 0)
        def _():
            pltpu.sync_copy(p_hbm, p_sh)

        pltpu.sync_copy(tok_hbm.at[pl.ds(chunk0, chunks_per_worker)], idx_v)
        plsc.subcore_barrier()

        def out_at(j):
            return out_hbm.at[pl.ds((chunk0 + j) * CHUNK, CHUNK)]

        @pl.loop(0, chunks_per_worker, step=4)
        def _(j):
            descs = []
            for pair in range(2):
                @pl.when(j >= 4)
                def _():
                    for b in range(2):
                        pltpu.make_async_copy(
                            rows[2 * pair + b], out_at(0), wsems[pair]
                        ).wait()
                descs.append([
                    pltpu.async_copy(
                        p_sh.at[idx_v.at[j + 2 * pair + b]],
                        rows[2 * pair + b],
                        gsems[pair],
                    )
                    for b in range(2)
                ])
            for pair in range(2):
                for b in range(2):
                    descs[pair][b].wait()
                    pltpu.async_copy(
                        rows[2 * pair + b], out_at(j + 2 * pair + b), wsems[pair]
                    )

        for pair in range(2):
            for b in range(2):
                pltpu.make_async_copy(rows[2 * pair + b], out_at(0), wsems[pair]).wait()

    return gather_kernel


def kernel(tokens, embedding_table, W, b):
    k, fan_in = embedding_table.shape
    t_total = tokens.shape[0]
    p = pl.pallas_call(
        _project_table_body,
        out_shape=jax.ShapeDtypeStruct((k, D_MODEL), jnp.float32),
    )(embedding_table, W, b.reshape(1, D_MODEL))
    tok2d = tokens.astype(jnp.int32).reshape(t_total // CHUNK, CHUNK)
    return _make_gather(t_total)(p, tok2d)
